# Initial kernel scaffold; baseline (speedup 1.0000x reference)
import functools

import jax
import jax.numpy as jnp
from jax import lax
from jax.experimental import pallas as pl
from jax.experimental.pallas import tpu as pltpu
from jax.experimental.pallas import tpu_sc as plsc

N = 10000
E = 320000
D = 128
H = 128
NT = 6

NC = 2
NS = 16
NW = NC * NS
EPW = E // NW
CHUNK = 80
NCHUNK = EPW // CHUNK
RPT = N // NS

ROWB = 1000
GRID = N // ROWB


def _sigmoid(x):
    return 1.0 / (1.0 + jnp.exp(-x))


def _embed_body(x_ref, w_ref, b_ref, g_ref, h_ref, tab_ref):
    h = jnp.maximum(
        jnp.dot(x_ref[...], w_ref[...], preferred_element_type=jnp.float32)
        + b_ref[...], 0.0)
    h_ref[...] = h
    for t in range(NT):
        gt = _sigmoid(g_ref[t, 0])
        tab_ref[t] = gt * h


def _embed_call(x, w_in, b_in, gate):
    return pl.pallas_call(
        _embed_body,
        grid=(GRID,),
        in_specs=[
            pl.BlockSpec((ROWB, D), lambda i: (i, 0)),
            pl.BlockSpec((D, H), lambda i: (0, 0)),
            pl.BlockSpec((1, H), lambda i: (0, 0)),
            pl.BlockSpec((NT, 1), lambda i: (0, 0), memory_space=pltpu.SMEM),
        ],
        out_specs=[
            pl.BlockSpec((ROWB, H), lambda i: (i, 0)),
            pl.BlockSpec((NT, ROWB, H), lambda i: (0, i, 0)),
        ],
        out_shape=[
            jax.ShapeDtypeStruct((N, H), jnp.float32),
            jax.ShapeDtypeStruct((NT, N, H), jnp.float32),
        ],
    )(x, w_in, b_in, gate)


def _sc_body_deg(tab, srcs, dsts, ets, zf, zd, ones_h,
                 p_out, dp_out,
                 acc, dacc, srcb, etb, dstb, idxb, rows, onesb, gsem):
    c = lax.axis_index("c")
    s = lax.axis_index("s")
    base = (s * NC + c) * EPW
    r0 = s * RPT
    pltpu.sync_copy(zf.at[pl.ds(r0, RPT)], acc.at[pl.ds(r0, RPT)])
    pltpu.sync_copy(zd.at[pl.ds(r0, RPT)], dacc.at[pl.ds(r0, RPT)])
    pltpu.sync_copy(ones_h, onesb)
    plsc.subcore_barrier()

    def step(i, carry):
        off = base + i * CHUNK
        pltpu.sync_copy(srcs.at[pl.ds(off, CHUNK)], srcb)
        pltpu.sync_copy(ets.at[pl.ds(off, CHUNK)], etb)
        pltpu.sync_copy(dsts.at[pl.ds(off, CHUNK)], dstb)
        for j in range(CHUNK // 16):
            sl = pl.ds(j * 16, 16)
            idxb[sl] = etb[sl] * N + srcb[sl]
        pltpu.async_copy(tab.at[idxb], rows, gsem).wait()
        pltpu.sync_copy(rows, acc.at[dstb], add=True)
        pltpu.sync_copy(onesb, dacc.at[dstb], add=True)
        return carry

    lax.fori_loop(0, NCHUNK, step, 0)
    plsc.subcore_barrier()
    pltpu.sync_copy(acc.at[pl.ds(r0, RPT)], p_out.at[c, pl.ds(r0, RPT)])
    pltpu.sync_copy(dacc.at[pl.ds(r0, RPT)], dp_out.at[c, pl.ds(r0, RPT)])


def _sc_body(tab, srcs, dsts, ets, zf,
             p_out,
             acc, srcb, etb, dstb, idxb, rows, gsem):
    c = lax.axis_index("c")
    s = lax.axis_index("s")
    base = (s * NC + c) * EPW
    r0 = s * RPT
    pltpu.sync_copy(zf.at[pl.ds(r0, RPT)], acc.at[pl.ds(r0, RPT)])
    plsc.subcore_barrier()

    def step(i, carry):
        off = base + i * CHUNK
        pltpu.sync_copy(srcs.at[pl.ds(off, CHUNK)], srcb)
        pltpu.sync_copy(ets.at[pl.ds(off, CHUNK)], etb)
        pltpu.sync_copy(dsts.at[pl.ds(off, CHUNK)], dstb)
        for j in range(CHUNK // 16):
            sl = pl.ds(j * 16, 16)
            idxb[sl] = etb[sl] * N + srcb[sl]
        pltpu.async_copy(tab.at[idxb], rows, gsem).wait()
        pltpu.sync_copy(rows, acc.at[dstb], add=True)
        return carry

    lax.fori_loop(0, NCHUNK, step, 0)
    plsc.subcore_barrier()
    pltpu.sync_copy(acc.at[pl.ds(r0, RPT)], p_out.at[c, pl.ds(r0, RPT)])


_MESH = plsc.VectorSubcoreMesh(core_axis_name="c", subcore_axis_name="s")

_sc_layer0 = pl.kernel(
    _sc_body_deg,
    out_type=[
        jax.ShapeDtypeStruct((NC, N, D), jnp.float32),
        jax.ShapeDtypeStruct((NC, N, 16), jnp.float32),
    ],
    mesh=_MESH,
    scratch_types=[
        pltpu.VMEM_SHARED((N, D), jnp.float32),
        pltpu.VMEM_SHARED((N, 16), jnp.float32),
        pltpu.VMEM((CHUNK,), jnp.int32),
        pltpu.VMEM((CHUNK,), jnp.int32),
        pltpu.VMEM((CHUNK,), jnp.int32),
        pltpu.VMEM((CHUNK,), jnp.int32),
        pltpu.VMEM((CHUNK, D), jnp.float32),
        pltpu.VMEM((CHUNK, 16), jnp.float32),
        pltpu.SemaphoreType.DMA,
    ],
)

_sc_layer1 = pl.kernel(
    _sc_body,
    out_type=[jax.ShapeDtypeStruct((NC, N, D), jnp.float32)],
    mesh=_MESH,
    scratch_types=[
        pltpu.VMEM_SHARED((N, D), jnp.float32),
        pltpu.VMEM((CHUNK,), jnp.int32),
        pltpu.VMEM((CHUNK,), jnp.int32),
        pltpu.VMEM((CHUNK,), jnp.int32),
        pltpu.VMEM((CHUNK,), jnp.int32),
        pltpu.VMEM((CHUNK, D), jnp.float32),
        pltpu.SemaphoreType.DMA,
    ],
)


def _mid_body(h_ref, p_ref, dg_ref, ws_ref, bs_ref, wn_ref, bn_ref, g_ref,
              hn_ref, tab_ref):
    d = dg_ref[...]
    deg = jnp.maximum(d[0, :, 0:1] + d[1, :, 0:1], 1.0)
    p = p_ref[...]
    agg = (p[0] + p[1]) / deg
    hn = jnp.maximum(
        jnp.dot(h_ref[...], ws_ref[...], preferred_element_type=jnp.float32)
        + jnp.dot(agg, wn_ref[...], preferred_element_type=jnp.float32)
        + bs_ref[...] + bn_ref[...], 0.0)
    hn_ref[...] = hn
    for t in range(NT):
        gt = _sigmoid(g_ref[t, 0])
        tab_ref[t] = gt * hn


def _mid_call(h, p, dg, ws, bs, wn, bn, gate):
    return pl.pallas_call(
        _mid_body,
        grid=(GRID,),
        in_specs=[
            pl.BlockSpec((ROWB, H), lambda i: (i, 0)),
            pl.BlockSpec((NC, ROWB, D), lambda i: (0, i, 0)),
            pl.BlockSpec((NC, ROWB, 16), lambda i: (0, i, 0)),
            pl.BlockSpec((H, H), lambda i: (0, 0)),
            pl.BlockSpec((1, H), lambda i: (0, 0)),
            pl.BlockSpec((H, H), lambda i: (0, 0)),
            pl.BlockSpec((1, H), lambda i: (0, 0)),
            pl.BlockSpec((NT, 1), lambda i: (0, 0), memory_space=pltpu.SMEM),
        ],
        out_specs=[
            pl.BlockSpec((ROWB, H), lambda i: (i, 0)),
            pl.BlockSpec((NT, ROWB, H), lambda i: (0, i, 0)),
        ],
        out_shape=[
            jax.ShapeDtypeStruct((N, H), jnp.float32),
            jax.ShapeDtypeStruct((NT, N, H), jnp.float32),
        ],
    )(h, p, dg, ws, bs, wn, bn, gate)


def _final_body(h_ref, p_ref, dg_ref, ws_ref, bs_ref, wn_ref, bn_ref,
                w1_ref, b1_ref, w2_ref, b2_ref, out_ref):
    d = dg_ref[...]
    deg = jnp.maximum(d[0, :, 0:1] + d[1, :, 0:1], 1.0)
    p = p_ref[...]
    agg = (p[0] + p[1]) / deg
    hn = jnp.maximum(
        jnp.dot(h_ref[...], ws_ref[...], preferred_element_type=jnp.float32)
        + jnp.dot(agg, wn_ref[...], preferred_element_type=jnp.float32)
        + bs_ref[...] + bn_ref[...], 0.0)
    z = jnp.maximum(
        jnp.dot(hn, w1_ref[...], preferred_element_type=jnp.float32)
        + b1_ref[...], 0.0)
    out_ref[...] = _sigmoid(
        jnp.dot(z, w2_ref[...], preferred_element_type=jnp.float32)
        + b2_ref[...])


def _final_call(h, p, dg, ws, bs, wn, bn, w1, b1, w2, b2):
    return pl.pallas_call(
        _final_body,
        grid=(GRID,),
        in_specs=[
            pl.BlockSpec((ROWB, H), lambda i: (i, 0)),
            pl.BlockSpec((NC, ROWB, D), lambda i: (0, i, 0)),
            pl.BlockSpec((NC, ROWB, 16), lambda i: (0, i, 0)),
            pl.BlockSpec((H, H), lambda i: (0, 0)),
            pl.BlockSpec((1, H), lambda i: (0, 0)),
            pl.BlockSpec((H, H), lambda i: (0, 0)),
            pl.BlockSpec((1, H), lambda i: (0, 0)),
            pl.BlockSpec((H, H // 2), lambda i: (0, 0)),
            pl.BlockSpec((1, H // 2), lambda i: (0, 0)),
            pl.BlockSpec((H // 2, 1), lambda i: (0, 0)),
            pl.BlockSpec((1, 1), lambda i: (0, 0)),
        ],
        out_specs=pl.BlockSpec((ROWB, 1), lambda i: (i, 0)),
        out_shape=jax.ShapeDtypeStruct((N, 1), jnp.float32),
    )(h, p, dg, ws, bs, wn, bn, w1, b1, w2, b2)


def kernel(x, edge_index, edge_type, W_in, b_in, W_self0, b_self0, W_neigh0,
           b_neigh0, W_self1, b_self1, W_neigh1, b_neigh1, gate_table, W_h1,
           b_h1, W_h2, b_h2):
    src = edge_index[0].astype(jnp.int32)
    dst = edge_index[1].astype(jnp.int32)
    et = edge_type.astype(jnp.int32)

    zf = jnp.zeros((N, D), jnp.float32)
    zd = jnp.zeros((N, 16), jnp.float32)
    ones_h = jnp.ones((CHUNK, 16), jnp.float32)

    h0, tab0 = _embed_call(x, W_in, b_in.reshape(1, H), gate_table)
    p0, dp = _sc_layer0(tab0.reshape(NT * N, D), src, dst, et, zf, zd, ones_h)
    h1, tab1 = _mid_call(h0, p0, dp, W_self0, b_self0.reshape(1, H),
                         W_neigh0, b_neigh0.reshape(1, H), gate_table)
    (p1,) = _sc_layer1(tab1.reshape(NT * N, D), src, dst, et, zf)
    out = _final_call(h1, p1, dp, W_self1, b_self1.reshape(1, H),
                      W_neigh1, b_neigh1.reshape(1, H),
                      W_h1, b_h1.reshape(1, H // 2), W_h2, b_h2.reshape(1, 1))
    return out.reshape(N)


# speedup vs baseline: 3.8923x; 3.8923x over previous
import jax
import jax.numpy as jnp
from jax import lax
from jax.experimental import pallas as pl
from jax.experimental.pallas import tpu as pltpu
from jax.experimental.pallas import tpu_sc as plsc

N = 10000
E = 320000
D = 128
H = 128
NT = 6

NC = 2
NS = 16
NW = NC * NS
EPW = E // NW
CHUNK = 80
NCHUNK = EPW // CHUNK
RPT = 1000
NINIT = N // RPT
ZROWS = 40

ROWB = 1000
GRID = N // ROWB


def _sigmoid(x):
    return 1.0 / (1.0 + jnp.exp(-x))


def _embed_body(x_ref, w_ref, b_ref, g_ref, h_ref, tab_ref):
    h = jnp.maximum(
        jnp.dot(x_ref[...], w_ref[...], preferred_element_type=jnp.float32)
        + b_ref[...], 0.0)
    h_ref[...] = h
    for t in range(NT):
        gt = _sigmoid(g_ref[t, 0])
        tab_ref[t] = gt * h


def _embed_call(x, w_in, b_in, gate):
    return pl.pallas_call(
        _embed_body,
        grid=(GRID,),
        in_specs=[
            pl.BlockSpec((ROWB, D), lambda i: (i, 0)),
            pl.BlockSpec((D, H), lambda i: (0, 0)),
            pl.BlockSpec((1, H), lambda i: (0, 0)),
            pl.BlockSpec((NT, 1), lambda i: (0, 0), memory_space=pltpu.SMEM),
        ],
        out_specs=[
            pl.BlockSpec((ROWB, H), lambda i: (i, 0)),
            pl.BlockSpec((NT, ROWB, H), lambda i: (0, i, 0)),
        ],
        out_shape=[
            jax.ShapeDtypeStruct((N, H), jnp.float32),
            jax.ShapeDtypeStruct((NT, N, H), jnp.float32),
        ],
    )(x, w_in, b_in, gate)


def _sc_edge_body(tab, srcs, dsts, ets, zf,
                  p_out,
                  acc, srcb, etb, dstb, idxb, rows, zbuf, gsem):
    c = lax.axis_index("c")
    s = lax.axis_index("s")
    base = (s * NC + c) * EPW
    r0 = s * RPT

    @pl.when(s < NINIT)
    def _init():
        pltpu.sync_copy(zf.at[pl.ds(0, ZROWS)], zbuf)
        for k in range(RPT // ZROWS):
            pltpu.sync_copy(zbuf, acc.at[pl.ds(r0 + k * ZROWS, ZROWS)])

    plsc.subcore_barrier()

    def step(i, carry):
        off = base + i * CHUNK
        pltpu.sync_copy(srcs.at[pl.ds(off, CHUNK)], srcb)
        pltpu.sync_copy(ets.at[pl.ds(off, CHUNK)], etb)
        pltpu.sync_copy(dsts.at[pl.ds(off, CHUNK)], dstb)
        for j in range(CHUNK // 16):
            sl = pl.ds(j * 16, 16)
            idxb[sl] = etb[sl] * N + srcb[sl]
        pltpu.async_copy(tab.at[idxb], rows, gsem).wait()
        pltpu.sync_copy(rows, acc.at[dstb], add=True)
        return carry

    lax.fori_loop(0, NCHUNK, step, 0)
    plsc.subcore_barrier()

    @pl.when(s < NINIT)
    def _writeout():
        for k in range(RPT // ZROWS):
            rk = r0 + k * ZROWS
            pltpu.sync_copy(acc.at[pl.ds(rk, ZROWS)], zbuf)
            pltpu.sync_copy(zbuf, p_out.at[c, pl.ds(rk, ZROWS)])


_sc_edge = pl.kernel(
    _sc_edge_body,
    out_type=[jax.ShapeDtypeStruct((NC, N, D), jnp.float32)],
    mesh=plsc.VectorSubcoreMesh(core_axis_name="c", subcore_axis_name="s"),
    scratch_types=[
        pltpu.VMEM_SHARED((N, D), jnp.float32),
        pltpu.VMEM((CHUNK,), jnp.int32),
        pltpu.VMEM((CHUNK,), jnp.int32),
        pltpu.VMEM((CHUNK,), jnp.int32),
        pltpu.VMEM((CHUNK,), jnp.int32),
        pltpu.VMEM((CHUNK, D), jnp.float32),
        pltpu.VMEM((ZROWS, D), jnp.float32),
        pltpu.SemaphoreType.DMA,
    ],
)


def _sc_deg_body(dsts, zf, ones_h,
                 dp_out,
                 acc, dstb, onesb, zbuf, gsem):
    c = lax.axis_index("c")
    s = lax.axis_index("s")
    base = (s * NC + c) * EPW
    r0 = s * RPT

    pltpu.sync_copy(ones_h, onesb)

    @pl.when(s < NINIT)
    def _init():
        pltpu.sync_copy(zf.at[pl.ds(0, ZROWS)], zbuf)
        for k in range(RPT // ZROWS):
            pltpu.sync_copy(zbuf, acc.at[pl.ds(r0 + k * ZROWS, ZROWS)])

    plsc.subcore_barrier()

    def step(i, carry):
        off = base + i * CHUNK
        pltpu.sync_copy(dsts.at[pl.ds(off, CHUNK)], dstb)
        pltpu.sync_copy(onesb, acc.at[dstb], add=True)
        return carry

    lax.fori_loop(0, NCHUNK, step, 0)
    plsc.subcore_barrier()

    @pl.when(s < NINIT)
    def _writeout():
        for k in range(RPT // ZROWS):
            rk = r0 + k * ZROWS
            pltpu.sync_copy(acc.at[pl.ds(rk, ZROWS)], zbuf)
            pltpu.sync_copy(zbuf, dp_out.at[c, pl.ds(rk, ZROWS)])


_sc_deg = pl.kernel(
    _sc_deg_body,
    out_type=[jax.ShapeDtypeStruct((NC, N, D), jnp.float32)],
    mesh=plsc.VectorSubcoreMesh(core_axis_name="c", subcore_axis_name="s"),
    scratch_types=[
        pltpu.VMEM_SHARED((N, D), jnp.float32),
        pltpu.VMEM((CHUNK,), jnp.int32),
        pltpu.VMEM((CHUNK, D), jnp.float32),
        pltpu.VMEM((ZROWS, D), jnp.float32),
        pltpu.SemaphoreType.DMA,
    ],
)


def _mid_body(h_ref, p_ref, dg_ref, ws_ref, bs_ref, wn_ref, bn_ref, g_ref,
              hn_ref, tab_ref):
    d = dg_ref[...]
    deg = jnp.maximum(d[0, :, 0:1] + d[1, :, 0:1], 1.0)
    p = p_ref[...]
    agg = (p[0] + p[1]) / deg
    hn = jnp.maximum(
        jnp.dot(h_ref[...], ws_ref[...], preferred_element_type=jnp.float32)
        + jnp.dot(agg, wn_ref[...], preferred_element_type=jnp.float32)
        + bs_ref[...] + bn_ref[...], 0.0)
    hn_ref[...] = hn
    for t in range(NT):
        gt = _sigmoid(g_ref[t, 0])
        tab_ref[t] = gt * hn


def _mid_call(h, p, dg, ws, bs, wn, bn, gate):
    return pl.pallas_call(
        _mid_body,
        grid=(GRID,),
        in_specs=[
            pl.BlockSpec((ROWB, H), lambda i: (i, 0)),
            pl.BlockSpec((NC, ROWB, D), lambda i: (0, i, 0)),
            pl.BlockSpec((NC, ROWB, D), lambda i: (0, i, 0)),
            pl.BlockSpec((H, H), lambda i: (0, 0)),
            pl.BlockSpec((1, H), lambda i: (0, 0)),
            pl.BlockSpec((H, H), lambda i: (0, 0)),
            pl.BlockSpec((1, H), lambda i: (0, 0)),
            pl.BlockSpec((NT, 1), lambda i: (0, 0), memory_space=pltpu.SMEM),
        ],
        out_specs=[
            pl.BlockSpec((ROWB, H), lambda i: (i, 0)),
            pl.BlockSpec((NT, ROWB, H), lambda i: (0, i, 0)),
        ],
        out_shape=[
            jax.ShapeDtypeStruct((N, H), jnp.float32),
            jax.ShapeDtypeStruct((NT, N, H), jnp.float32),
        ],
    )(h, p, dg, ws, bs, wn, bn, gate)


def _final_body(h_ref, p_ref, dg_ref, ws_ref, bs_ref, wn_ref, bn_ref,
                w1_ref, b1_ref, w2_ref, b2_ref, out_ref):
    d = dg_ref[...]
    deg = jnp.maximum(d[0, :, 0:1] + d[1, :, 0:1], 1.0)
    p = p_ref[...]
    agg = (p[0] + p[1]) / deg
    hn = jnp.maximum(
        jnp.dot(h_ref[...], ws_ref[...], preferred_element_type=jnp.float32)
        + jnp.dot(agg, wn_ref[...], preferred_element_type=jnp.float32)
        + bs_ref[...] + bn_ref[...], 0.0)
    z = jnp.maximum(
        jnp.dot(hn, w1_ref[...], preferred_element_type=jnp.float32)
        + b1_ref[...], 0.0)
    out_ref[...] = _sigmoid(
        jnp.dot(z, w2_ref[...], preferred_element_type=jnp.float32)
        + b2_ref[...])


def _final_call(h, p, dg, ws, bs, wn, bn, w1, b1, w2, b2):
    return pl.pallas_call(
        _final_body,
        grid=(GRID,),
        in_specs=[
            pl.BlockSpec((ROWB, H), lambda i: (i, 0)),
            pl.BlockSpec((NC, ROWB, D), lambda i: (0, i, 0)),
            pl.BlockSpec((NC, ROWB, D), lambda i: (0, i, 0)),
            pl.BlockSpec((H, H), lambda i: (0, 0)),
            pl.BlockSpec((1, H), lambda i: (0, 0)),
            pl.BlockSpec((H, H), lambda i: (0, 0)),
            pl.BlockSpec((1, H), lambda i: (0, 0)),
            pl.BlockSpec((H, H // 2), lambda i: (0, 0)),
            pl.BlockSpec((1, H // 2), lambda i: (0, 0)),
            pl.BlockSpec((H // 2, 1), lambda i: (0, 0)),
            pl.BlockSpec((1, 1), lambda i: (0, 0)),
        ],
        out_specs=pl.BlockSpec((ROWB, 1), lambda i: (i, 0)),
        out_shape=jax.ShapeDtypeStruct((N, 1), jnp.float32),
    )(h, p, dg, ws, bs, wn, bn, w1, b1, w2, b2)


def kernel(x, edge_index, edge_type, W_in, b_in, W_self0, b_self0, W_neigh0,
           b_neigh0, W_self1, b_self1, W_neigh1, b_neigh1, gate_table, W_h1,
           b_h1, W_h2, b_h2):
    src = edge_index[0].astype(jnp.int32)
    dst = edge_index[1].astype(jnp.int32)
    et = edge_type.astype(jnp.int32)

    zf = jnp.zeros((N, D), jnp.float32)
    ones_h = jnp.ones((CHUNK, D), jnp.float32)

    h0, tab0 = _embed_call(x, W_in, b_in.reshape(1, H), gate_table)
    (dp,) = _sc_deg(dst, zf, ones_h)
    (p0,) = _sc_edge(tab0.reshape(NT * N, D), src, dst, et, zf)
    h1, tab1 = _mid_call(h0, p0, dp, W_self0, b_self0.reshape(1, H),
                         W_neigh0, b_neigh0.reshape(1, H), gate_table)
    (p1,) = _sc_edge(tab1.reshape(NT * N, D), src, dst, et, zf)
    out = _final_call(h1, p1, dp, W_self1, b_self1.reshape(1, H),
                      W_neigh1, b_neigh1.reshape(1, H),
                      W_h1, b_h1.reshape(1, H // 2), W_h2, b_h2.reshape(1, 1))
    return out.reshape(N)


# speedup vs baseline: 6.7622x; 1.7373x over previous
import jax
import jax.numpy as jnp
from jax import lax
from jax.experimental import pallas as pl
from jax.experimental.pallas import tpu as pltpu
from jax.experimental.pallas import tpu_sc as plsc

N = 10000
E = 320000
D = 128
H = 128
NT = 6

NC = 2
NS = 16
NW = NC * NS
EPW = E // NW
CHUNK = 80
NCHUNK = EPW // CHUNK
RPT = 1000
NINIT = N // RPT
ZROWS = 40

ROWB = 1000
GRID = N // ROWB


def _sigmoid(x):
    return 1.0 / (1.0 + jnp.exp(-x))


def _embed_body(x_ref, w_ref, b_ref, g_ref, h_ref, tab_ref):
    h = jnp.maximum(
        jnp.dot(x_ref[...], w_ref[...], preferred_element_type=jnp.float32)
        + b_ref[...], 0.0)
    h_ref[...] = h
    for t in range(NT):
        gt = _sigmoid(g_ref[t, 0])
        tab_ref[t] = gt * h


def _embed_call(x, w_in, b_in, gate):
    return pl.pallas_call(
        _embed_body,
        grid=(GRID,),
        in_specs=[
            pl.BlockSpec((ROWB, D), lambda i: (i, 0)),
            pl.BlockSpec((D, H), lambda i: (0, 0)),
            pl.BlockSpec((1, H), lambda i: (0, 0)),
            pl.BlockSpec((NT, 1), lambda i: (0, 0), memory_space=pltpu.SMEM),
        ],
        out_specs=[
            pl.BlockSpec((ROWB, H), lambda i: (i, 0)),
            pl.BlockSpec((NT, ROWB, H), lambda i: (0, i, 0)),
        ],
        out_shape=[
            jax.ShapeDtypeStruct((N, H), jnp.float32),
            jax.ShapeDtypeStruct((NT, N, H), jnp.float32),
        ],
    )(x, w_in, b_in, gate)


def _sc_edge_body(tab, srcs, dsts, ets, zf,
                  p_out,
                  acc, srcb, etb, dstb, idxb, rows, zbuf, isem, gsem, ssem):
    c = lax.axis_index("c")
    s = lax.axis_index("s")
    base = (s * NC + c) * EPW
    r0 = s * RPT

    @pl.when(s < NINIT)
    def _init():
        pltpu.sync_copy(zf.at[pl.ds(0, ZROWS)], zbuf)
        for k in range(RPT // ZROWS):
            pltpu.sync_copy(zbuf, acc.at[pl.ds(r0 + k * ZROWS, ZROWS)])

    plsc.subcore_barrier()

    def issue_idx(off, p):
        pltpu.async_copy(srcs.at[pl.ds(off, CHUNK)], srcb.at[p], isem)
        pltpu.async_copy(ets.at[pl.ds(off, CHUNK)], etb.at[p], isem)
        pltpu.async_copy(dsts.at[pl.ds(off, CHUNK)], dstb.at[p], isem)

    def wait_idx(p):
        for b in (srcb, etb, dstb):
            pltpu.make_async_copy(srcs.at[pl.ds(0, CHUNK)], b.at[p],
                                  isem).wait()

    def wait_scatter(p):
        pltpu.make_async_copy(rows.at[p], acc.at[dstb.at[p]], ssem).wait()

    def half(i, p, first, last):
        wait_idx(p)
        for j in range(CHUNK // 16):
            sl = pl.ds(j * 16, 16)
            idxb[p, sl] = etb[p, sl] * N + srcb[p, sl]
        g = pltpu.async_copy(tab.at[idxb.at[p]], rows.at[p], gsem)
        if not first:
            wait_scatter(1 - p)
        if not last:
            issue_idx(base + (i + 1) * CHUNK, 1 - p)
        g.wait()
        pltpu.async_copy(rows.at[p], acc.at[dstb.at[p]], ssem, add=True)

    issue_idx(base, 0)
    half(0, 0, True, False)

    def step(k, carry):
        i = 1 + 2 * k
        half(i, 1, False, False)
        half(i + 1, 0, False, False)
        return carry

    lax.fori_loop(0, (NCHUNK - 3) // 2, step, 0)
    half(NCHUNK - 2, 1, False, False)
    half(NCHUNK - 1, 0, False, True)
    wait_scatter(0)

    plsc.subcore_barrier()

    @pl.when(s < NINIT)
    def _writeout():
        for k in range(RPT // ZROWS):
            rk = r0 + k * ZROWS
            pltpu.sync_copy(acc.at[pl.ds(rk, ZROWS)], zbuf)
            pltpu.sync_copy(zbuf, p_out.at[c, pl.ds(rk, ZROWS)])


_sc_edge = pl.kernel(
    _sc_edge_body,
    out_type=[jax.ShapeDtypeStruct((NC, N, D), jnp.float32)],
    mesh=plsc.VectorSubcoreMesh(core_axis_name="c", subcore_axis_name="s"),
    scratch_types=[
        pltpu.VMEM_SHARED((N, D), jnp.float32),
        pltpu.VMEM((2, CHUNK), jnp.int32),
        pltpu.VMEM((2, CHUNK), jnp.int32),
        pltpu.VMEM((2, CHUNK), jnp.int32),
        pltpu.VMEM((2, CHUNK), jnp.int32),
        pltpu.VMEM((2, CHUNK, D), jnp.float32),
        pltpu.VMEM((ZROWS, D), jnp.float32),
        pltpu.SemaphoreType.DMA,
        pltpu.SemaphoreType.DMA,
        pltpu.SemaphoreType.DMA,
    ],
)


def _sc_deg_body(dsts, zf, ones_h,
                 dp_out,
                 acc, dstb, onesb, zbuf, gsem):
    c = lax.axis_index("c")
    s = lax.axis_index("s")
    base = (s * NC + c) * EPW
    r0 = s * RPT

    pltpu.sync_copy(ones_h, onesb)

    @pl.when(s < NINIT)
    def _init():
        pltpu.sync_copy(zf.at[pl.ds(0, ZROWS)], zbuf)
        for k in range(RPT // ZROWS):
            pltpu.sync_copy(zbuf, acc.at[pl.ds(r0 + k * ZROWS, ZROWS)])

    plsc.subcore_barrier()

    def step(i, carry):
        off = base + i * CHUNK
        pltpu.sync_copy(dsts.at[pl.ds(off, CHUNK)], dstb)
        pltpu.sync_copy(onesb, acc.at[dstb], add=True)
        return carry

    lax.fori_loop(0, NCHUNK, step, 0)
    plsc.subcore_barrier()

    @pl.when(s < NINIT)
    def _writeout():
        for k in range(RPT // ZROWS):
            rk = r0 + k * ZROWS
            pltpu.sync_copy(acc.at[pl.ds(rk, ZROWS)], zbuf)
            pltpu.sync_copy(zbuf, dp_out.at[c, pl.ds(rk, ZROWS)])


_sc_deg = pl.kernel(
    _sc_deg_body,
    out_type=[jax.ShapeDtypeStruct((NC, N, D), jnp.float32)],
    mesh=plsc.VectorSubcoreMesh(core_axis_name="c", subcore_axis_name="s"),
    scratch_types=[
        pltpu.VMEM_SHARED((N, D), jnp.float32),
        pltpu.VMEM((CHUNK,), jnp.int32),
        pltpu.VMEM((CHUNK, D), jnp.float32),
        pltpu.VMEM((ZROWS, D), jnp.float32),
        pltpu.SemaphoreType.DMA,
    ],
)


def _mid_body(h_ref, p_ref, dg_ref, ws_ref, bs_ref, wn_ref, bn_ref, g_ref,
              hn_ref, tab_ref):
    d = dg_ref[...]
    deg = jnp.maximum(d[0, :, 0:1] + d[1, :, 0:1], 1.0)
    p = p_ref[...]
    agg = (p[0] + p[1]) / deg
    hn = jnp.maximum(
        jnp.dot(h_ref[...], ws_ref[...], preferred_element_type=jnp.float32)
        + jnp.dot(agg, wn_ref[...], preferred_element_type=jnp.float32)
        + bs_ref[...] + bn_ref[...], 0.0)
    hn_ref[...] = hn
    for t in range(NT):
        gt = _sigmoid(g_ref[t, 0])
        tab_ref[t] = gt * hn


def _mid_call(h, p, dg, ws, bs, wn, bn, gate):
    return pl.pallas_call(
        _mid_body,
        grid=(GRID,),
        in_specs=[
            pl.BlockSpec((ROWB, H), lambda i: (i, 0)),
            pl.BlockSpec((NC, ROWB, D), lambda i: (0, i, 0)),
            pl.BlockSpec((NC, ROWB, D), lambda i: (0, i, 0)),
            pl.BlockSpec((H, H), lambda i: (0, 0)),
            pl.BlockSpec((1, H), lambda i: (0, 0)),
            pl.BlockSpec((H, H), lambda i: (0, 0)),
            pl.BlockSpec((1, H), lambda i: (0, 0)),
            pl.BlockSpec((NT, 1), lambda i: (0, 0), memory_space=pltpu.SMEM),
        ],
        out_specs=[
            pl.BlockSpec((ROWB, H), lambda i: (i, 0)),
            pl.BlockSpec((NT, ROWB, H), lambda i: (0, i, 0)),
        ],
        out_shape=[
            jax.ShapeDtypeStruct((N, H), jnp.float32),
            jax.ShapeDtypeStruct((NT, N, H), jnp.float32),
        ],
    )(h, p, dg, ws, bs, wn, bn, gate)


def _final_body(h_ref, p_ref, dg_ref, ws_ref, bs_ref, wn_ref, bn_ref,
                w1_ref, b1_ref, w2_ref, b2_ref, out_ref):
    d = dg_ref[...]
    deg = jnp.maximum(d[0, :, 0:1] + d[1, :, 0:1], 1.0)
    p = p_ref[...]
    agg = (p[0] + p[1]) / deg
    hn = jnp.maximum(
        jnp.dot(h_ref[...], ws_ref[...], preferred_element_type=jnp.float32)
        + jnp.dot(agg, wn_ref[...], preferred_element_type=jnp.float32)
        + bs_ref[...] + bn_ref[...], 0.0)
    z = jnp.maximum(
        jnp.dot(hn, w1_ref[...], preferred_element_type=jnp.float32)
        + b1_ref[...], 0.0)
    out_ref[...] = _sigmoid(
        jnp.dot(z, w2_ref[...], preferred_element_type=jnp.float32)
        + b2_ref[...])


def _final_call(h, p, dg, ws, bs, wn, bn, w1, b1, w2, b2):
    return pl.pallas_call(
        _final_body,
        grid=(GRID,),
        in_specs=[
            pl.BlockSpec((ROWB, H), lambda i: (i, 0)),
            pl.BlockSpec((NC, ROWB, D), lambda i: (0, i, 0)),
            pl.BlockSpec((NC, ROWB, D), lambda i: (0, i, 0)),
            pl.BlockSpec((H, H), lambda i: (0, 0)),
            pl.BlockSpec((1, H), lambda i: (0, 0)),
            pl.BlockSpec((H, H), lambda i: (0, 0)),
            pl.BlockSpec((1, H), lambda i: (0, 0)),
            pl.BlockSpec((H, H // 2), lambda i: (0, 0)),
            pl.BlockSpec((1, H // 2), lambda i: (0, 0)),
            pl.BlockSpec((H // 2, 1), lambda i: (0, 0)),
            pl.BlockSpec((1, 1), lambda i: (0, 0)),
        ],
        out_specs=pl.BlockSpec((ROWB, 1), lambda i: (i, 0)),
        out_shape=jax.ShapeDtypeStruct((N, 1), jnp.float32),
    )(h, p, dg, ws, bs, wn, bn, w1, b1, w2, b2)


def kernel(x, edge_index, edge_type, W_in, b_in, W_self0, b_self0, W_neigh0,
           b_neigh0, W_self1, b_self1, W_neigh1, b_neigh1, gate_table, W_h1,
           b_h1, W_h2, b_h2):
    src = edge_index[0].astype(jnp.int32)
    dst = edge_index[1].astype(jnp.int32)
    et = edge_type.astype(jnp.int32)

    zf = jnp.zeros((N, D), jnp.float32)
    ones_h = jnp.ones((CHUNK, D), jnp.float32)

    h0, tab0 = _embed_call(x, W_in, b_in.reshape(1, H), gate_table)
    (dp,) = _sc_deg(dst, zf, ones_h)
    (p0,) = _sc_edge(tab0.reshape(NT * N, D), src, dst, et, zf)
    h1, tab1 = _mid_call(h0, p0, dp, W_self0, b_self0.reshape(1, H),
                         W_neigh0, b_neigh0.reshape(1, H), gate_table)
    (p1,) = _sc_edge(tab1.reshape(NT * N, D), src, dst, et, zf)
    out = _final_call(h1, p1, dp, W_self1, b_self1.reshape(1, H),
                      W_neigh1, b_neigh1.reshape(1, H),
                      W_h1, b_h1.reshape(1, H // 2), W_h2, b_h2.reshape(1, 1))
    return out.reshape(N)


# speedup vs baseline: 8.0280x; 1.1872x over previous
import jax
import jax.numpy as jnp
from jax import lax
from jax.experimental import pallas as pl
from jax.experimental.pallas import tpu as pltpu
from jax.experimental.pallas import tpu_sc as plsc

N = 10000
E = 320000
D = 128
H = 128
NT = 6

NC = 2
NS = 16
NW = NC * NS
EPW = E // NW
CHUNK = 80
NCHUNK = EPW // CHUNK
RPT = 1000
NINIT = N // RPT
ZROWS = 40

ROWB = 1000
GRID = N // ROWB


def _sigmoid(x):
    return 1.0 / (1.0 + jnp.exp(-x))


def _embed_body(x_ref, w_ref, b_ref, g_ref, h_ref, tab_ref):
    h = jnp.maximum(
        jnp.dot(x_ref[...], w_ref[...], preferred_element_type=jnp.float32)
        + b_ref[...], 0.0)
    h_ref[...] = h
    for t in range(NT):
        gt = _sigmoid(g_ref[t, 0])
        tab_ref[t] = gt * h


def _embed_call(x, w_in, b_in, gate):
    return pl.pallas_call(
        _embed_body,
        grid=(GRID,),
        in_specs=[
            pl.BlockSpec((ROWB, D), lambda i: (i, 0)),
            pl.BlockSpec((D, H), lambda i: (0, 0)),
            pl.BlockSpec((1, H), lambda i: (0, 0)),
            pl.BlockSpec((NT, 1), lambda i: (0, 0), memory_space=pltpu.SMEM),
        ],
        out_specs=[
            pl.BlockSpec((ROWB, H), lambda i: (i, 0)),
            pl.BlockSpec((NT, ROWB, H), lambda i: (0, i, 0)),
        ],
        out_shape=[
            jax.ShapeDtypeStruct((N, H), jnp.float32),
            jax.ShapeDtypeStruct((NT, N, H), jnp.float32),
        ],
    )(x, w_in, b_in, gate)


def _sc_edge_body(tab, srcs, dsts, ets, zf,
                  p_out,
                  acc, srcb, etb, dstb, idxb, rows, zbuf, isem, gsem, ssem):
    c = lax.axis_index("c")
    s = lax.axis_index("s")
    base = (s * NC + c) * EPW
    r0 = s * RPT

    @pl.when(s < NINIT)
    def _init():
        pltpu.sync_copy(zf.at[pl.ds(0, ZROWS)], zbuf)
        for k in range(RPT // ZROWS):
            pltpu.sync_copy(zbuf, acc.at[pl.ds(r0 + k * ZROWS, ZROWS)])

    plsc.subcore_barrier()

    def issue_idx(off, p):
        pltpu.async_copy(srcs.at[pl.ds(off, CHUNK)], srcb.at[p], isem)
        pltpu.async_copy(ets.at[pl.ds(off, CHUNK)], etb.at[p], isem)
        pltpu.async_copy(dsts.at[pl.ds(off, CHUNK)], dstb.at[p], isem)

    def wait_idx(p):
        for b in (srcb, etb, dstb):
            pltpu.make_async_copy(srcs.at[pl.ds(0, CHUNK)], b.at[p],
                                  isem).wait()

    def wait_scatter(p):
        pltpu.make_async_copy(rows.at[p], acc.at[dstb.at[p]], ssem).wait()

    def half(i, p, first, last):
        wait_idx(p)
        for j in range(CHUNK // 16):
            sl = pl.ds(j * 16, 16)
            idxb[p, sl] = etb[p, sl] * N + srcb[p, sl]
        g = pltpu.async_copy(tab.at[idxb.at[p]], rows.at[p], gsem)
        if not first:
            wait_scatter(1 - p)
        if not last:
            issue_idx(base + (i + 1) * CHUNK, 1 - p)
        g.wait()
        pltpu.async_copy(rows.at[p], acc.at[dstb.at[p]], ssem, add=True)

    issue_idx(base, 0)
    half(0, 0, True, False)

    def step(k, carry):
        i = 1 + 2 * k
        half(i, 1, False, False)
        half(i + 1, 0, False, False)
        return carry

    lax.fori_loop(0, (NCHUNK - 3) // 2, step, 0)
    half(NCHUNK - 2, 1, False, False)
    half(NCHUNK - 1, 0, False, True)
    wait_scatter(0)

    plsc.subcore_barrier()

    @pl.when(s < NINIT)
    def _writeout():
        for k in range(RPT // ZROWS):
            rk = r0 + k * ZROWS
            pltpu.sync_copy(acc.at[pl.ds(rk, ZROWS)], zbuf)
            pltpu.sync_copy(zbuf, p_out.at[c, pl.ds(rk, ZROWS)])


_sc_edge = pl.kernel(
    _sc_edge_body,
    out_type=[jax.ShapeDtypeStruct((NC, N, D), jnp.float32)],
    mesh=plsc.VectorSubcoreMesh(core_axis_name="c", subcore_axis_name="s"),
    scratch_types=[
        pltpu.VMEM_SHARED((N, D), jnp.float32),
        pltpu.VMEM((2, CHUNK), jnp.int32),
        pltpu.VMEM((2, CHUNK), jnp.int32),
        pltpu.VMEM((2, CHUNK), jnp.int32),
        pltpu.VMEM((2, CHUNK), jnp.int32),
        pltpu.VMEM((2, CHUNK, D), jnp.float32),
        pltpu.VMEM((ZROWS, D), jnp.float32),
        pltpu.SemaphoreType.DMA,
        pltpu.SemaphoreType.DMA,
        pltpu.SemaphoreType.DMA,
    ],
)


def _sc_deg_body(dsts, z1,
                 dp_out,
                 degv, dstb, isem):
    c = lax.axis_index("c")
    s = lax.axis_index("s")
    w = s * NC + c
    base = w * EPW

    pltpu.sync_copy(z1, degv)

    def issue_idx(off, p):
        pltpu.async_copy(dsts.at[pl.ds(off, CHUNK)], dstb.at[p], isem)

    def wait_idx(p):
        pltpu.make_async_copy(dsts.at[pl.ds(0, CHUNK)], dstb.at[p],
                              isem).wait()

    def half(i, p, last):
        wait_idx(p)
        if not last:
            issue_idx(base + (i + 1) * CHUNK, 1 - p)
        for j in range(CHUNK // 16):
            idx16 = dstb.at[p][pl.ds(j * 16, 16)]
            ones16 = ((idx16 - idx16) + 1).astype(jnp.float32)
            plsc.addupdate_scatter(degv, [idx16], ones16)

    issue_idx(base, 0)

    def step(k, carry):
        half(2 * k, 0, False)
        half(2 * k + 1, 1, False)
        return carry

    lax.fori_loop(0, (NCHUNK - 1) // 2, step, 0)
    half(NCHUNK - 1, 0, True)

    pltpu.sync_copy(degv, dp_out.at[w])


_sc_deg = pl.kernel(
    _sc_deg_body,
    out_type=[jax.ShapeDtypeStruct((NW, N), jnp.float32)],
    mesh=plsc.VectorSubcoreMesh(core_axis_name="c", subcore_axis_name="s"),
    compiler_params=pltpu.CompilerParams(needs_layout_passes=False),
    scratch_types=[
        pltpu.VMEM((N,), jnp.float32),
        pltpu.VMEM((2, CHUNK), jnp.int32),
        pltpu.SemaphoreType.DMA,
    ],
)


def _mid_body(h_ref, p_ref, dg_ref, ws_ref, bs_ref, wn_ref, bn_ref, g_ref,
              hn_ref, tab_ref):
    deg = jnp.maximum(jnp.sum(dg_ref[...], axis=1, keepdims=True), 1.0)
    p = p_ref[...]
    agg = (p[0] + p[1]) / deg
    hn = jnp.maximum(
        jnp.dot(h_ref[...], ws_ref[...], preferred_element_type=jnp.float32)
        + jnp.dot(agg, wn_ref[...], preferred_element_type=jnp.float32)
        + bs_ref[...] + bn_ref[...], 0.0)
    hn_ref[...] = hn
    for t in range(NT):
        gt = _sigmoid(g_ref[t, 0])
        tab_ref[t] = gt * hn


def _mid_call(h, p, dg, ws, bs, wn, bn, gate):
    return pl.pallas_call(
        _mid_body,
        grid=(GRID,),
        in_specs=[
            pl.BlockSpec((ROWB, H), lambda i: (i, 0)),
            pl.BlockSpec((NC, ROWB, D), lambda i: (0, i, 0)),
            pl.BlockSpec((ROWB, NW), lambda i: (i, 0)),
            pl.BlockSpec((H, H), lambda i: (0, 0)),
            pl.BlockSpec((1, H), lambda i: (0, 0)),
            pl.BlockSpec((H, H), lambda i: (0, 0)),
            pl.BlockSpec((1, H), lambda i: (0, 0)),
            pl.BlockSpec((NT, 1), lambda i: (0, 0), memory_space=pltpu.SMEM),
        ],
        out_specs=[
            pl.BlockSpec((ROWB, H), lambda i: (i, 0)),
            pl.BlockSpec((NT, ROWB, H), lambda i: (0, i, 0)),
        ],
        out_shape=[
            jax.ShapeDtypeStruct((N, H), jnp.float32),
            jax.ShapeDtypeStruct((NT, N, H), jnp.float32),
        ],
    )(h, p, dg, ws, bs, wn, bn, gate)


def _final_body(h_ref, p_ref, dg_ref, ws_ref, bs_ref, wn_ref, bn_ref,
                w1_ref, b1_ref, w2_ref, b2_ref, out_ref):
    deg = jnp.maximum(jnp.sum(dg_ref[...], axis=1, keepdims=True), 1.0)
    p = p_ref[...]
    agg = (p[0] + p[1]) / deg
    hn = jnp.maximum(
        jnp.dot(h_ref[...], ws_ref[...], preferred_element_type=jnp.float32)
        + jnp.dot(agg, wn_ref[...], preferred_element_type=jnp.float32)
        + bs_ref[...] + bn_ref[...], 0.0)
    z = jnp.maximum(
        jnp.dot(hn, w1_ref[...], preferred_element_type=jnp.float32)
        + b1_ref[...], 0.0)
    out_ref[...] = _sigmoid(
        jnp.dot(z, w2_ref[...], preferred_element_type=jnp.float32)
        + b2_ref[...])


def _final_call(h, p, dg, ws, bs, wn, bn, w1, b1, w2, b2):
    return pl.pallas_call(
        _final_body,
        grid=(GRID,),
        in_specs=[
            pl.BlockSpec((ROWB, H), lambda i: (i, 0)),
            pl.BlockSpec((NC, ROWB, D), lambda i: (0, i, 0)),
            pl.BlockSpec((ROWB, NW), lambda i: (i, 0)),
            pl.BlockSpec((H, H), lambda i: (0, 0)),
            pl.BlockSpec((1, H), lambda i: (0, 0)),
            pl.BlockSpec((H, H), lambda i: (0, 0)),
            pl.BlockSpec((1, H), lambda i: (0, 0)),
            pl.BlockSpec((H, H // 2), lambda i: (0, 0)),
            pl.BlockSpec((1, H // 2), lambda i: (0, 0)),
            pl.BlockSpec((H // 2, 1), lambda i: (0, 0)),
            pl.BlockSpec((1, 1), lambda i: (0, 0)),
        ],
        out_specs=pl.BlockSpec((ROWB, 1), lambda i: (i, 0)),
        out_shape=jax.ShapeDtypeStruct((N, 1), jnp.float32),
    )(h, p, dg, ws, bs, wn, bn, w1, b1, w2, b2)


def kernel(x, edge_index, edge_type, W_in, b_in, W_self0, b_self0, W_neigh0,
           b_neigh0, W_self1, b_self1, W_neigh1, b_neigh1, gate_table, W_h1,
           b_h1, W_h2, b_h2):
    src = edge_index[0].astype(jnp.int32)
    dst = edge_index[1].astype(jnp.int32)
    et = edge_type.astype(jnp.int32)

    zf = jnp.zeros((N, D), jnp.float32)
    z1 = jnp.zeros((N,), jnp.float32)

    h0, tab0 = _embed_call(x, W_in, b_in.reshape(1, H), gate_table)
    (dp,) = _sc_deg(dst, z1)
    (p0,) = _sc_edge(tab0.reshape(NT * N, D), src, dst, et, zf)
    dpt = dp.T
    h1, tab1 = _mid_call(h0, p0, dpt, W_self0, b_self0.reshape(1, H),
                         W_neigh0, b_neigh0.reshape(1, H), gate_table)
    (p1,) = _sc_edge(tab1.reshape(NT * N, D), src, dst, et, zf)
    out = _final_call(h1, p1, dpt, W_self1, b_self1.reshape(1, H),
                      W_neigh1, b_neigh1.reshape(1, H),
                      W_h1, b_h1.reshape(1, H // 2), W_h2, b_h2.reshape(1, 1))
    return out.reshape(N)


# speedup vs baseline: 9.0261x; 1.1243x over previous
import jax
import jax.numpy as jnp
from jax import lax
from jax.experimental import pallas as pl
from jax.experimental.pallas import tpu as pltpu
from jax.experimental.pallas import tpu_sc as plsc

N = 10000
E = 320000
D = 128
H = 128
NT = 6

NC = 2
NS = 16
NW = NC * NS
EPW = E // NW
CHUNK = 80
NCHUNK = EPW // CHUNK
EC = 128
NEC = EPW // EC
ETAIL = EPW - NEC * EC
RPT = 1000
NINIT = N // RPT
ZROWS = 40

ROWB = 1000
GRID = N // ROWB


def _sigmoid(x):
    return 1.0 / (1.0 + jnp.exp(-x))


def _embed_body(x_ref, w_ref, b_ref, g_ref, h_ref, tab_ref):
    h = jnp.maximum(
        jnp.dot(x_ref[...], w_ref[...], preferred_element_type=jnp.float32)
        + b_ref[...], 0.0)
    h_ref[...] = h
    for t in range(NT):
        gt = _sigmoid(g_ref[t, 0])
        tab_ref[t] = gt * h


def _embed_call(x, w_in, b_in, gate):
    return pl.pallas_call(
        _embed_body,
        grid=(GRID,),
        in_specs=[
            pl.BlockSpec((ROWB, D), lambda i: (i, 0)),
            pl.BlockSpec((D, H), lambda i: (0, 0)),
            pl.BlockSpec((1, H), lambda i: (0, 0)),
            pl.BlockSpec((NT, 1), lambda i: (0, 0), memory_space=pltpu.SMEM),
        ],
        out_specs=[
            pl.BlockSpec((ROWB, H), lambda i: (i, 0)),
            pl.BlockSpec((NT, ROWB, H), lambda i: (0, i, 0)),
        ],
        out_shape=[
            jax.ShapeDtypeStruct((N, H), jnp.float32),
            jax.ShapeDtypeStruct((NT, N, H), jnp.float32),
        ],
    )(x, w_in, b_in, gate)


def _sc_edge_body(tab, srcs, dsts, ets, zf,
                  p_out,
                  acc, srcb, etb, dstb, idxb, rows, zbuf,
                  srct, ett, dstt, idxt, rowst, isem, gsem, ssem):
    c = lax.axis_index("c")
    s = lax.axis_index("s")
    base = (s * NC + c) * EPW
    r0 = s * RPT

    @pl.when(s < NINIT)
    def _init():
        pltpu.sync_copy(zf.at[pl.ds(0, ZROWS)], zbuf)
        for k in range(RPT // ZROWS):
            pltpu.sync_copy(zbuf, acc.at[pl.ds(r0 + k * ZROWS, ZROWS)])

    plsc.subcore_barrier()

    def issue_idx(off, p):
        pltpu.async_copy(srcs.at[pl.ds(off, EC)], srcb.at[p], isem)
        pltpu.async_copy(ets.at[pl.ds(off, EC)], etb.at[p], isem)
        pltpu.async_copy(dsts.at[pl.ds(off, EC)], dstb.at[p], isem)

    def wait_idx(p):
        for b in (srcb, etb, dstb):
            pltpu.make_async_copy(srcs.at[pl.ds(0, EC)], b.at[p],
                                  isem).wait()

    def wait_scatter(p):
        pltpu.make_async_copy(rows.at[p], acc.at[dstb.at[p]], ssem).wait()

    def half(i, p, first, last):
        wait_idx(p)
        for j in range(EC // 16):
            sl = pl.ds(j * 16, 16)
            idxb[p, sl] = etb[p, sl] * N + srcb[p, sl]
        g = pltpu.async_copy(tab.at[idxb.at[p]], rows.at[p], gsem)
        if not first:
            wait_scatter(1 - p)
        if not last:
            issue_idx(base + (i + 1) * EC, 1 - p)
        g.wait()
        pltpu.async_copy(rows.at[p], acc.at[dstb.at[p]], ssem, add=True)

    issue_idx(base, 0)
    half(0, 0, True, False)

    def step(k, carry):
        i = 1 + 2 * k
        half(i, 1, False, False)
        half(i + 1, 0, False, False)
        return carry

    lax.fori_loop(0, (NEC - 2) // 2, step, 0)
    half(NEC - 1, 1, False, True)
    wait_scatter(1)

    toff = base + NEC * EC
    pltpu.sync_copy(srcs.at[pl.ds(toff, ETAIL)], srct)
    pltpu.sync_copy(ets.at[pl.ds(toff, ETAIL)], ett)
    pltpu.sync_copy(dsts.at[pl.ds(toff, ETAIL)], dstt)
    idxt[...] = ett[...] * N + srct[...]
    pltpu.async_copy(tab.at[idxt], rowst, gsem).wait()
    pltpu.sync_copy(rowst, acc.at[dstt], add=True)

    plsc.subcore_barrier()

    @pl.when(s < NINIT)
    def _writeout():
        for k in range(RPT // ZROWS):
            rk = r0 + k * ZROWS
            pltpu.sync_copy(acc.at[pl.ds(rk, ZROWS)], zbuf)
            pltpu.sync_copy(zbuf, p_out.at[c, pl.ds(rk, ZROWS)])


_sc_edge = pl.kernel(
    _sc_edge_body,
    out_type=[jax.ShapeDtypeStruct((NC, N, D), jnp.float32)],
    mesh=plsc.VectorSubcoreMesh(core_axis_name="c", subcore_axis_name="s"),
    scratch_types=[
        pltpu.VMEM_SHARED((N, D), jnp.float32),
        pltpu.VMEM((2, EC), jnp.int32),
        pltpu.VMEM((2, EC), jnp.int32),
        pltpu.VMEM((2, EC), jnp.int32),
        pltpu.VMEM((2, EC), jnp.int32),
        pltpu.VMEM((2, EC, D), jnp.float32),
        pltpu.VMEM((ZROWS, D), jnp.float32),
        pltpu.VMEM((ETAIL,), jnp.int32),
        pltpu.VMEM((ETAIL,), jnp.int32),
        pltpu.VMEM((ETAIL,), jnp.int32),
        pltpu.VMEM((ETAIL,), jnp.int32),
        pltpu.VMEM((ETAIL, D), jnp.float32),
        pltpu.SemaphoreType.DMA,
        pltpu.SemaphoreType.DMA,
        pltpu.SemaphoreType.DMA,
    ],
)


def _sc_deg_body(dsts, z1,
                 dp_out,
                 degv, dstb, isem):
    c = lax.axis_index("c")
    s = lax.axis_index("s")
    w = s * NC + c
    base = w * EPW

    pltpu.sync_copy(z1, degv)

    def issue_idx(off, p):
        pltpu.async_copy(dsts.at[pl.ds(off, CHUNK)], dstb.at[p], isem)

    def wait_idx(p):
        pltpu.make_async_copy(dsts.at[pl.ds(0, CHUNK)], dstb.at[p],
                              isem).wait()

    def half(i, p, last):
        wait_idx(p)
        if not last:
            issue_idx(base + (i + 1) * CHUNK, 1 - p)
        for j in range(CHUNK // 16):
            idx16 = dstb.at[p][pl.ds(j * 16, 16)]
            ones16 = ((idx16 - idx16) + 1).astype(jnp.float32)
            plsc.addupdate_scatter(degv, [idx16], ones16)

    issue_idx(base, 0)

    def step(k, carry):
        half(2 * k, 0, False)
        half(2 * k + 1, 1, False)
        return carry

    lax.fori_loop(0, (NCHUNK - 1) // 2, step, 0)
    half(NCHUNK - 1, 0, True)

    pltpu.sync_copy(degv, dp_out.at[w])


_sc_deg = pl.kernel(
    _sc_deg_body,
    out_type=[jax.ShapeDtypeStruct((NW, N), jnp.float32)],
    mesh=plsc.VectorSubcoreMesh(core_axis_name="c", subcore_axis_name="s"),
    compiler_params=pltpu.CompilerParams(needs_layout_passes=False),
    scratch_types=[
        pltpu.VMEM((N,), jnp.float32),
        pltpu.VMEM((2, CHUNK), jnp.int32),
        pltpu.SemaphoreType.DMA,
    ],
)


def _mid_body(h_ref, p_ref, dg_ref, ws_ref, bs_ref, wn_ref, bn_ref, g_ref,
              hn_ref, tab_ref):
    deg = jnp.maximum(jnp.sum(dg_ref[...], axis=1, keepdims=True), 1.0)
    p = p_ref[...]
    agg = (p[0] + p[1]) / deg
    hn = jnp.maximum(
        jnp.dot(h_ref[...], ws_ref[...], preferred_element_type=jnp.float32)
        + jnp.dot(agg, wn_ref[...], preferred_element_type=jnp.float32)
        + bs_ref[...] + bn_ref[...], 0.0)
    hn_ref[...] = hn
    for t in range(NT):
        gt = _sigmoid(g_ref[t, 0])
        tab_ref[t] = gt * hn


def _mid_call(h, p, dg, ws, bs, wn, bn, gate):
    return pl.pallas_call(
        _mid_body,
        grid=(GRID,),
        in_specs=[
            pl.BlockSpec((ROWB, H), lambda i: (i, 0)),
            pl.BlockSpec((NC, ROWB, D), lambda i: (0, i, 0)),
            pl.BlockSpec((ROWB, NW), lambda i: (i, 0)),
            pl.BlockSpec((H, H), lambda i: (0, 0)),
            pl.BlockSpec((1, H), lambda i: (0, 0)),
            pl.BlockSpec((H, H), lambda i: (0, 0)),
            pl.BlockSpec((1, H), lambda i: (0, 0)),
            pl.BlockSpec((NT, 1), lambda i: (0, 0), memory_space=pltpu.SMEM),
        ],
        out_specs=[
            pl.BlockSpec((ROWB, H), lambda i: (i, 0)),
            pl.BlockSpec((NT, ROWB, H), lambda i: (0, i, 0)),
        ],
        out_shape=[
            jax.ShapeDtypeStruct((N, H), jnp.float32),
            jax.ShapeDtypeStruct((NT, N, H), jnp.float32),
        ],
    )(h, p, dg, ws, bs, wn, bn, gate)


def _final_body(h_ref, p_ref, dg_ref, ws_ref, bs_ref, wn_ref, bn_ref,
                w1_ref, b1_ref, w2_ref, b2_ref, out_ref):
    deg = jnp.maximum(jnp.sum(dg_ref[...], axis=1, keepdims=True), 1.0)
    p = p_ref[...]
    agg = (p[0] + p[1]) / deg
    hn = jnp.maximum(
        jnp.dot(h_ref[...], ws_ref[...], preferred_element_type=jnp.float32)
        + jnp.dot(agg, wn_ref[...], preferred_element_type=jnp.float32)
        + bs_ref[...] + bn_ref[...], 0.0)
    z = jnp.maximum(
        jnp.dot(hn, w1_ref[...], preferred_element_type=jnp.float32)
        + b1_ref[...], 0.0)
    out_ref[...] = _sigmoid(
        jnp.dot(z, w2_ref[...], preferred_element_type=jnp.float32)
        + b2_ref[...])


def _final_call(h, p, dg, ws, bs, wn, bn, w1, b1, w2, b2):
    return pl.pallas_call(
        _final_body,
        grid=(GRID,),
        in_specs=[
            pl.BlockSpec((ROWB, H), lambda i: (i, 0)),
            pl.BlockSpec((NC, ROWB, D), lambda i: (0, i, 0)),
            pl.BlockSpec((ROWB, NW), lambda i: (i, 0)),
            pl.BlockSpec((H, H), lambda i: (0, 0)),
            pl.BlockSpec((1, H), lambda i: (0, 0)),
            pl.BlockSpec((H, H), lambda i: (0, 0)),
            pl.BlockSpec((1, H), lambda i: (0, 0)),
            pl.BlockSpec((H, H // 2), lambda i: (0, 0)),
            pl.BlockSpec((1, H // 2), lambda i: (0, 0)),
            pl.BlockSpec((H // 2, 1), lambda i: (0, 0)),
            pl.BlockSpec((1, 1), lambda i: (0, 0)),
        ],
        out_specs=pl.BlockSpec((ROWB, 1), lambda i: (i, 0)),
        out_shape=jax.ShapeDtypeStruct((N, 1), jnp.float32),
    )(h, p, dg, ws, bs, wn, bn, w1, b1, w2, b2)


def kernel(x, edge_index, edge_type, W_in, b_in, W_self0, b_self0, W_neigh0,
           b_neigh0, W_self1, b_self1, W_neigh1, b_neigh1, gate_table, W_h1,
           b_h1, W_h2, b_h2):
    src = edge_index[0].astype(jnp.int32)
    dst = edge_index[1].astype(jnp.int32)
    et = edge_type.astype(jnp.int32)

    zf = jnp.zeros((N, D), jnp.float32)
    z1 = jnp.zeros((N,), jnp.float32)

    h0, tab0 = _embed_call(x, W_in, b_in.reshape(1, H), gate_table)
    (dp,) = _sc_deg(dst, z1)
    (p0,) = _sc_edge(tab0.reshape(NT * N, D), src, dst, et, zf)
    dpt = dp.T
    h1, tab1 = _mid_call(h0, p0, dpt, W_self0, b_self0.reshape(1, H),
                         W_neigh0, b_neigh0.reshape(1, H), gate_table)
    (p1,) = _sc_edge(tab1.reshape(NT * N, D), src, dst, et, zf)
    out = _final_call(h1, p1, dpt, W_self1, b_self1.reshape(1, H),
                      W_neigh1, b_neigh1.reshape(1, H),
                      W_h1, b_h1.reshape(1, H // 2), W_h2, b_h2.reshape(1, 1))
    return out.reshape(N)
